# baseline (speedup 1.0000x reference)
import functools

import jax
import jax.numpy as jnp
from jax.experimental import pallas as pl

RATIO = 16


def _body(mask_ref, emb_ref, out_ref, flag_ref):
    flags = jnp.all(mask_ref[...], axis=1)
    flag_ref[0, 0, :] = flags
    keep = 1.0 - flags.astype(jnp.float32)
    out_ref[...] = keep[:, None] * emb_ref[1, :][None, :]


@functools.lru_cache(maxsize=None)
def _make_tc_call(n: int, d: int, blk: int):
    grid = n // blk
    return pl.pallas_call(
        _body,
        grid=(grid,),
        in_specs=[
            pl.BlockSpec((blk, RATIO), lambda i: (i, 0)),
            pl.BlockSpec((2, d), lambda i: (0, 0)),
        ],
        out_specs=[
            pl.BlockSpec((blk, d), lambda i: (i, 0)),
            pl.BlockSpec((1, 1, blk), lambda i: (i, 0, 0)),
        ],
        out_shape=[
            jax.ShapeDtypeStruct((n, d), jnp.float32),
            jax.ShapeDtypeStruct((grid, 1, blk), jnp.bool_),
        ],
    )


def kernel(token_ids, padding_mask, embeds):
    B, Sl = padding_mask.shape
    d = embeds.shape[1]
    Sg = Sl // RATIO
    n = B * Sg
    out_flat, flags = _make_tc_call(n, d, 2048)(
        padding_mask.reshape(n, RATIO), embeds)
    return out_flat.reshape(B, Sg, d), flags.reshape(B, Sg)


# speedup vs baseline: 1.0447x; 1.0447x over previous
import functools

import jax
import jax.numpy as jnp
from jax.experimental import pallas as pl

RATIO = 16


def _body(mask_ref, emb_ref, out_ref, flag_ref):
    flags = jnp.all(mask_ref[...], axis=1)
    flag_ref[0, 0, :] = flags
    keep = 1.0 - flags.astype(jnp.float32)
    out_ref[...] = keep[:, None] * emb_ref[1, :][None, :]


@functools.lru_cache(maxsize=None)
def _make_tc_call(n: int, d: int, blk: int):
    grid = n // blk
    return pl.pallas_call(
        _body,
        grid=(grid,),
        in_specs=[
            pl.BlockSpec((blk, RATIO), lambda i: (i, 0)),
            pl.BlockSpec((2, d), lambda i: (0, 0)),
        ],
        out_specs=[
            pl.BlockSpec((blk, d), lambda i: (i, 0)),
            pl.BlockSpec((1, 1, blk), lambda i: (i, 0, 0)),
        ],
        out_shape=[
            jax.ShapeDtypeStruct((n, d), jnp.float32),
            jax.ShapeDtypeStruct((grid, 1, blk), jnp.bool_),
        ],
    )


def kernel(token_ids, padding_mask, embeds):
    B, Sl = padding_mask.shape
    d = embeds.shape[1]
    Sg = Sl // RATIO
    n = B * Sg
    out_flat, flags = _make_tc_call(n, d, 1024)(
        padding_mask.reshape(n, RATIO), embeds)
    return out_flat.reshape(B, Sg, d), flags.reshape(B, Sg)
